# Initial kernel scaffold; baseline (speedup 1.0000x reference)
import functools

import jax
import jax.numpy as jnp
from jax import lax
from jax.experimental import pallas as pl
from jax.experimental.pallas import tpu as pltpu
from jax.experimental.pallas import tpu_sc as plsc

NC = 2
NS = 16
NW = NC * NS
CHUNK = 128


def _sc_segment_sum(n_pad, ch, feat, src3, dst3, zeros128, zeros16, ones16):
    rows_per_tile = n_pad // NS
    zcopies = rows_per_tile // 128

    mesh = plsc.VectorSubcoreMesh(
        core_axis_name="c", subcore_axis_name="s",
        num_cores=NC, num_subcores=NS)

    @functools.partial(
        pl.kernel,
        out_type=[
            jax.ShapeDtypeStruct((NC, n_pad, 128), jnp.float32),
            jax.ShapeDtypeStruct((NC, n_pad, 16), jnp.float32),
        ],
        mesh=mesh,
        scratch_types=[
            pltpu.VMEM((ch, CHUNK), jnp.int32),
            pltpu.VMEM((ch, CHUNK), jnp.int32),
            pltpu.VMEM((CHUNK, 128), jnp.float32),
            pltpu.VMEM((128, 128), jnp.float32),
            pltpu.VMEM((128, 16), jnp.float32),
            pltpu.VMEM((CHUNK, 16), jnp.float32),
            pltpu.VMEM_SHARED((n_pad, 128), jnp.float32),
            pltpu.VMEM_SHARED((n_pad, 16), jnp.float32),
            pltpu.SemaphoreType.DMA,
        ],
    )
    def sc_fn(feat_hbm, src_hbm, dst_hbm, z128_hbm, z16_hbm, o16_hbm,
              agg_out, deg_out,
              src_v, dst_v, rowbuf, zbuf, z16buf, onesbuf,
              agg_sp, deg_sp, sem):
        c = lax.axis_index("c")
        s = lax.axis_index("s")
        wid = s * NC + c

        pltpu.sync_copy(z128_hbm, zbuf)
        pltpu.sync_copy(z16_hbm, z16buf)
        pltpu.sync_copy(o16_hbm, onesbuf)
        pltpu.sync_copy(src_hbm.at[wid], src_v)
        pltpu.sync_copy(dst_hbm.at[wid], dst_v)

        base = s * rows_per_tile
        for r in range(zcopies):
            pltpu.sync_copy(zbuf, agg_sp.at[pl.ds(base + r * 128, 128)])
            pltpu.sync_copy(z16buf, deg_sp.at[pl.ds(base + r * 128, 128)])
        plsc.subcore_barrier()

        def body(j, carry):
            pltpu.async_copy(feat_hbm.at[src_v.at[j]], rowbuf, sem).wait()
            pltpu.sync_copy(rowbuf, agg_sp.at[dst_v.at[j]], add=True)
            pltpu.sync_copy(onesbuf, deg_sp.at[dst_v.at[j]], add=True)
            return carry

        lax.fori_loop(0, ch, body, 0)
        plsc.subcore_barrier()

        pltpu.sync_copy(agg_sp.at[pl.ds(base, rows_per_tile)],
                        agg_out.at[c, pl.ds(base, rows_per_tile)])
        pltpu.sync_copy(deg_sp.at[pl.ds(base, rows_per_tile)],
                        deg_out.at[c, pl.ds(base, rows_per_tile)])

    return sc_fn(feat, src3, dst3, zeros128, zeros16, ones16)


def _tc_epilogue(n_pad, agg0, agg1, deg0, deg1, W, b2d):
    blk = 1024

    def body(a0, a1, d0, d1, w, bv, o):
        acc = a0[...] + a1[...]
        deg = d0[...][:, 0:1] + d1[...][:, 0:1]
        o[...] = lax.dot_general(
            acc, w[...], (((1,), (1,)), ((), ())),
            preferred_element_type=jnp.float32) + deg * bv[...]

    return pl.pallas_call(
        body,
        grid=(n_pad // blk,),
        in_specs=[
            pl.BlockSpec((blk, 128), lambda i: (i, 0)),
            pl.BlockSpec((blk, 128), lambda i: (i, 0)),
            pl.BlockSpec((blk, 16), lambda i: (i, 0)),
            pl.BlockSpec((blk, 16), lambda i: (i, 0)),
            pl.BlockSpec((128, 128), lambda i: (0, 0)),
            pl.BlockSpec((1, 128), lambda i: (0, 0)),
        ],
        out_specs=pl.BlockSpec((blk, 128), lambda i: (i, 0)),
        out_shape=jax.ShapeDtypeStruct((n_pad, 128), jnp.float32),
    )(agg0, agg1, deg0, deg1, W, b2d)


def kernel(feat, edge_index, W, b):
    n = feat.shape[0]
    e = edge_index.shape[1]
    n_pad = ((n + 2047) // 2048) * 2048
    epw = CHUNK * (-(-e // (NW * CHUNK)))
    e_pad = NW * epw
    ch = epw // CHUNK

    src = edge_index[0].astype(jnp.int32)
    dst = edge_index[1].astype(jnp.int32)
    src3 = jnp.concatenate(
        [src, jnp.zeros((e_pad - e,), jnp.int32)]).reshape(NW, ch, CHUNK)
    dst3 = jnp.concatenate(
        [dst, jnp.full((e_pad - e,), n, jnp.int32)]).reshape(NW, ch, CHUNK)

    zeros128 = jnp.zeros((128, 128), jnp.float32)
    zeros16 = jnp.zeros((128, 16), jnp.float32)
    ones16 = jnp.ones((CHUNK, 16), jnp.float32)

    agg, deg = _sc_segment_sum(n_pad, ch, feat, src3, dst3,
                               zeros128, zeros16, ones16)
    out_full = _tc_epilogue(n_pad, agg[0], agg[1], deg[0], deg[1],
                            W, b.reshape(1, -1))
    return out_full[:n]


# speedup vs baseline: 5.3647x; 5.3647x over previous
import functools

import jax
import jax.numpy as jnp
from jax import lax
from jax.experimental import pallas as pl
from jax.experimental.pallas import tpu as pltpu
from jax.experimental.pallas import tpu_sc as plsc

NC = 2
NS = 16
CHUNK = 128
HF = 64


def _sc_segment_sum(n_pad, ch, featL, featR, src3, dst3,
                    zeros64, zeros16, ones16):
    rows_per_tile = n_pad // NS
    zcopies = rows_per_tile // 128

    mesh = plsc.VectorSubcoreMesh(
        core_axis_name="c", subcore_axis_name="s",
        num_cores=NC, num_subcores=NS)

    @functools.partial(
        pl.kernel,
        out_type=[
            jax.ShapeDtypeStruct((NC, n_pad, HF), jnp.float32),
            jax.ShapeDtypeStruct((NC, n_pad, 16), jnp.float32),
        ],
        mesh=mesh,
        scratch_types=[
            pltpu.VMEM((ch, CHUNK), jnp.int32),
            pltpu.VMEM((ch, CHUNK), jnp.int32),
            pltpu.VMEM((CHUNK, HF), jnp.float32),
            pltpu.VMEM((128, HF), jnp.float32),
            pltpu.VMEM((128, 16), jnp.float32),
            pltpu.VMEM((CHUNK, 16), jnp.float32),
            pltpu.VMEM_SHARED((n_pad, HF), jnp.float32),
            pltpu.VMEM_SHARED((n_pad, 16), jnp.float32),
            pltpu.SemaphoreType.DMA,
        ],
        compiler_params=pltpu.CompilerParams(use_tc_tiling_on_sc=False),
    )
    def sc_fn(featL_hbm, featR_hbm, src_hbm, dst_hbm,
              z64_hbm, z16_hbm, o16_hbm,
              agg_out, deg_out,
              src_v, dst_v, rowbuf, zbuf, z16buf, onesbuf,
              agg_sp, deg_sp, sem):
        c = lax.axis_index("c")
        s = lax.axis_index("s")

        pltpu.sync_copy(z64_hbm, zbuf)
        pltpu.sync_copy(z16_hbm, z16buf)
        pltpu.sync_copy(o16_hbm, onesbuf)
        pltpu.sync_copy(src_hbm.at[s], src_v)
        pltpu.sync_copy(dst_hbm.at[s], dst_v)

        base = s * rows_per_tile
        for r in range(zcopies):
            pltpu.sync_copy(zbuf, agg_sp.at[pl.ds(base + r * 128, 128)])
            pltpu.sync_copy(z16buf, deg_sp.at[pl.ds(base + r * 128, 128)])
        plsc.subcore_barrier()

        def run(feat_half):
            def body(j, carry):
                pltpu.async_copy(feat_half.at[src_v.at[j]], rowbuf, sem).wait()
                pltpu.sync_copy(rowbuf, agg_sp.at[dst_v.at[j]], add=True)
                pltpu.sync_copy(onesbuf, deg_sp.at[dst_v.at[j]], add=True)
                return carry
            lax.fori_loop(0, ch, body, 0)

        @pl.when(c == 0)
        def _():
            run(featL_hbm)

        @pl.when(c == 1)
        def _():
            run(featR_hbm)

        plsc.subcore_barrier()

        pltpu.sync_copy(agg_sp.at[pl.ds(base, rows_per_tile)],
                        agg_out.at[c, pl.ds(base, rows_per_tile)])
        pltpu.sync_copy(deg_sp.at[pl.ds(base, rows_per_tile)],
                        deg_out.at[c, pl.ds(base, rows_per_tile)])

    return sc_fn(featL, featR, src3, dst3, zeros64, zeros16, ones16)


def _tc_epilogue(n_pad, aggL, aggR, deg0, deg1, WL, WR, b2d):
    blk = 1024
    dn = (((1,), (1,)), ((), ()))

    def body(a0, a1, d0, d1, wl, wr, bv, o):
        deg = (d0[...][:, 0:1] + d1[...][:, 0:1]) * 0.5
        o[...] = (
            lax.dot_general(a0[...], wl[...], dn,
                            preferred_element_type=jnp.float32)
            + lax.dot_general(a1[...], wr[...], dn,
                              preferred_element_type=jnp.float32)
            + deg * bv[...])

    return pl.pallas_call(
        body,
        grid=(n_pad // blk,),
        in_specs=[
            pl.BlockSpec((blk, HF), lambda i: (i, 0)),
            pl.BlockSpec((blk, HF), lambda i: (i, 0)),
            pl.BlockSpec((blk, 16), lambda i: (i, 0)),
            pl.BlockSpec((blk, 16), lambda i: (i, 0)),
            pl.BlockSpec((128, HF), lambda i: (0, 0)),
            pl.BlockSpec((128, HF), lambda i: (0, 0)),
            pl.BlockSpec((1, 128), lambda i: (0, 0)),
        ],
        out_specs=pl.BlockSpec((blk, 128), lambda i: (i, 0)),
        out_shape=jax.ShapeDtypeStruct((n_pad, 128), jnp.float32),
    )(aggL, aggR, deg0, deg1, WL, WR, b2d)


def kernel(feat, edge_index, W, b):
    n = feat.shape[0]
    e = edge_index.shape[1]
    n_pad = ((n + 2047) // 2048) * 2048
    epw = CHUNK * (-(-e // (NS * CHUNK)))
    e_pad = NS * epw
    ch = epw // CHUNK

    src = edge_index[0].astype(jnp.int32)
    dst = edge_index[1].astype(jnp.int32)
    src3 = jnp.concatenate(
        [src, jnp.zeros((e_pad - e,), jnp.int32)]).reshape(NS, ch, CHUNK)
    dst3 = jnp.concatenate(
        [dst, jnp.full((e_pad - e,), n, jnp.int32)]).reshape(NS, ch, CHUNK)

    featL = feat[:, :HF]
    featR = feat[:, HF:]
    zeros64 = jnp.zeros((128, HF), jnp.float32)
    zeros16 = jnp.zeros((128, 16), jnp.float32)
    ones16 = jnp.ones((CHUNK, 16), jnp.float32)

    agg, deg = _sc_segment_sum(n_pad, ch, featL, featR, src3, dst3,
                               zeros64, zeros16, ones16)
    out_full = _tc_epilogue(n_pad, agg[0], agg[1], deg[0], deg[1],
                            W[:, :HF], W[:, HF:], b.reshape(1, -1))
    return out_full[:n]


# speedup vs baseline: 6.4982x; 1.2113x over previous
import functools

import jax
import jax.numpy as jnp
from jax import lax
from jax.experimental import pallas as pl
from jax.experimental.pallas import tpu as pltpu
from jax.experimental.pallas import tpu_sc as plsc

NC = 2
NS = 16
CHUNK = 128
HF = 64


def _sc_segment_sum(n_pad, ch, featL, featR, src3, dst3,
                    zeros64, zeros16, ones16):
    rows_per_tile = n_pad // NS
    zcopies = rows_per_tile // 128

    mesh = plsc.VectorSubcoreMesh(
        core_axis_name="c", subcore_axis_name="s",
        num_cores=NC, num_subcores=NS)

    @functools.partial(
        pl.kernel,
        out_type=[
            jax.ShapeDtypeStruct((NC, n_pad, HF), jnp.float32),
            jax.ShapeDtypeStruct((NC, n_pad, 16), jnp.float32),
        ],
        mesh=mesh,
        scratch_types=[
            pltpu.VMEM((ch, CHUNK), jnp.int32),
            pltpu.VMEM((ch, CHUNK), jnp.int32),
            pltpu.VMEM((CHUNK, HF), jnp.float32),
            pltpu.VMEM((CHUNK, HF), jnp.float32),
            pltpu.VMEM((128, HF), jnp.float32),
            pltpu.VMEM((128, 16), jnp.float32),
            pltpu.VMEM((CHUNK, 16), jnp.float32),
            pltpu.VMEM_SHARED((n_pad, HF), jnp.float32),
            pltpu.VMEM_SHARED((n_pad, 16), jnp.float32),
            pltpu.SemaphoreType.DMA,
            pltpu.SemaphoreType.DMA,
        ],
        compiler_params=pltpu.CompilerParams(use_tc_tiling_on_sc=False),
    )
    def sc_fn(featL_hbm, featR_hbm, src_hbm, dst_hbm,
              z64_hbm, z16_hbm, o16_hbm,
              agg_out, deg_out,
              src_v, dst_v, rowbuf0, rowbuf1, zbuf, z16buf, onesbuf,
              agg_sp, deg_sp, sem0, sem1):
        c = lax.axis_index("c")
        s = lax.axis_index("s")

        pltpu.sync_copy(z64_hbm, zbuf)
        pltpu.sync_copy(z16_hbm, z16buf)
        pltpu.sync_copy(o16_hbm, onesbuf)
        pltpu.sync_copy(src_hbm.at[s], src_v)
        pltpu.sync_copy(dst_hbm.at[s], dst_v)

        base = s * rows_per_tile
        for r in range(zcopies):
            pltpu.sync_copy(zbuf, agg_sp.at[pl.ds(base + r * 128, 128)])
            pltpu.sync_copy(z16buf, deg_sp.at[pl.ds(base + r * 128, 128)])
        plsc.subcore_barrier()

        def run(feat_half, deg_par):
            pltpu.async_copy(feat_half.at[src_v.at[0]], rowbuf0, sem0)
            pltpu.async_copy(feat_half.at[src_v.at[1]], rowbuf1, sem1)

            def body(g, carry):
                j0 = g * 2
                for par, buf, sem in ((0, rowbuf0, sem0), (1, rowbuf1, sem1)):
                    j = j0 + par
                    pltpu.make_async_copy(feat_half.at[src_v.at[j]],
                                          buf, sem).wait()
                    pltpu.sync_copy(buf, agg_sp.at[dst_v.at[j]], add=True)
                    if deg_par == par:
                        pltpu.sync_copy(onesbuf, deg_sp.at[dst_v.at[j]],
                                        add=True)

                    @pl.when(j + 2 < ch)
                    def _():
                        pltpu.async_copy(feat_half.at[src_v.at[j + 2]],
                                         buf, sem)
                return carry

            lax.fori_loop(0, ch // 2, body, 0)

        @pl.when(c == 0)
        def _():
            run(featL_hbm, 0)

        @pl.when(c == 1)
        def _():
            run(featR_hbm, 1)

        plsc.subcore_barrier()

        pltpu.sync_copy(agg_sp.at[pl.ds(base, rows_per_tile)],
                        agg_out.at[c, pl.ds(base, rows_per_tile)])
        pltpu.sync_copy(deg_sp.at[pl.ds(base, rows_per_tile)],
                        deg_out.at[c, pl.ds(base, rows_per_tile)])

    return sc_fn(featL, featR, src3, dst3, zeros64, zeros16, ones16)


def _tc_epilogue(n_pad, aggL, aggR, deg0, deg1, WL, WR, b2d):
    blk = 1024
    dn = (((1,), (1,)), ((), ()))

    def body(a0, a1, d0, d1, wl, wr, bv, o):
        deg = d0[...][:, 0:1] + d1[...][:, 0:1]
        o[...] = (
            lax.dot_general(a0[...], wl[...], dn,
                            preferred_element_type=jnp.float32)
            + lax.dot_general(a1[...], wr[...], dn,
                              preferred_element_type=jnp.float32)
            + deg * bv[...])

    return pl.pallas_call(
        body,
        grid=(n_pad // blk,),
        in_specs=[
            pl.BlockSpec((blk, HF), lambda i: (i, 0)),
            pl.BlockSpec((blk, HF), lambda i: (i, 0)),
            pl.BlockSpec((blk, 16), lambda i: (i, 0)),
            pl.BlockSpec((blk, 16), lambda i: (i, 0)),
            pl.BlockSpec((128, HF), lambda i: (0, 0)),
            pl.BlockSpec((128, HF), lambda i: (0, 0)),
            pl.BlockSpec((1, 128), lambda i: (0, 0)),
        ],
        out_specs=pl.BlockSpec((blk, 128), lambda i: (i, 0)),
        out_shape=jax.ShapeDtypeStruct((n_pad, 128), jnp.float32),
    )(aggL, aggR, deg0, deg1, WL, WR, b2d)


def kernel(feat, edge_index, W, b):
    n = feat.shape[0]
    e = edge_index.shape[1]
    n_pad = ((n + 2047) // 2048) * 2048
    epw = 2 * CHUNK * (-(-e // (NS * 2 * CHUNK)))
    e_pad = NS * epw
    ch = epw // CHUNK

    src = edge_index[0].astype(jnp.int32)
    dst = edge_index[1].astype(jnp.int32)
    src3 = jnp.concatenate(
        [src, jnp.zeros((e_pad - e,), jnp.int32)]).reshape(NS, ch, CHUNK)
    dst3 = jnp.concatenate(
        [dst, jnp.full((e_pad - e,), n, jnp.int32)]).reshape(NS, ch, CHUNK)

    featL = feat[:, :HF]
    featR = feat[:, HF:]
    zeros64 = jnp.zeros((128, HF), jnp.float32)
    zeros16 = jnp.zeros((128, 16), jnp.float32)
    ones16 = jnp.ones((CHUNK, 16), jnp.float32)

    agg, deg = _sc_segment_sum(n_pad, ch, featL, featR, src3, dst3,
                               zeros64, zeros16, ones16)
    out_full = _tc_epilogue(n_pad, agg[0], agg[1], deg[0], deg[1],
                            W[:, :HF], W[:, HF:], b.reshape(1, -1))
    return out_full[:n]
